# baseline (speedup 1.0000x reference)
import functools

import jax
import jax.numpy as jnp
from jax import lax
from jax.experimental import pallas as pl
from jax.experimental.pallas import tpu as pltpu
from jax.experimental.pallas import tpu_sc as plsc

N = 10000
E = 320000
D = 128
NPAD = 10240

NC = 2
NS = 16
NW = NC * NS

CHUNK = 128
PH = 2
KPP = 40
KPW = PH * KPP
EP = NW * KPW * CHUNK

ZCH = NPAD // NS

_MESH = dict(mesh=plsc.VectorSubcoreMesh(core_axis_name="c", subcore_axis_name="s"))
_f32 = jnp.float32


def _wid():
    c = lax.axis_index("c")
    s = lax.axis_index("s")
    return c, s, c * NS + s


def _fill_ones(ones_v):
    for i in range(CHUNK // 16):
        ones_v[pl.ds(i * 16, 16)] = jnp.full((16,), 1.0, _f32)


@functools.partial(
    pl.kernel,
    out_type=(jax.ShapeDtypeStruct((NC, NPAD), _f32),) * 2,
    scratch_types=[
        pltpu.VMEM((KPP, CHUNK), jnp.int32),
        pltpu.VMEM((CHUNK,), _f32),
        pltpu.VMEM_SHARED((NPAD,), _f32),
        pltpu.VMEM_SHARED((NPAD,), _f32),
        pltpu.SemaphoreType.DMA,
    ],
    **_MESH,
)
def _sc_degs(src1_hbm, src2_hbm, z_hbm, cnt1_hbm, cnt2_hbm,
             idx_v, ones_v, acc1, acc2, ssem):
    c, s, w = _wid()
    _fill_ones(ones_v)
    zb = s * ZCH
    pltpu.sync_copy(z_hbm.at[pl.ds(zb, ZCH)], acc1.at[pl.ds(zb, ZCH)])
    pltpu.sync_copy(z_hbm.at[pl.ds(zb, ZCH)], acc2.at[pl.ds(zb, ZCH)])
    plsc.subcore_barrier()

    def fire(k, _, acc=None):
        pltpu.async_copy(ones_v, acc.at[idx_v.at[k]], ssem, add=True)
        return ()

    def drain(k, _, acc=None):
        pltpu.make_async_copy(ones_v, acc.at[idx_v.at[0]], ssem).wait()
        return ()

    for src_hbm, acc in ((src1_hbm, acc1), (src2_hbm, acc2)):
        for ph in range(PH):
            pltpu.sync_copy(src_hbm.at[w].at[ph], idx_v)
            lax.fori_loop(0, KPP, functools.partial(fire, acc=acc), ())
            lax.fori_loop(0, KPP, functools.partial(drain, acc=acc), ())

    plsc.subcore_barrier()
    pltpu.sync_copy(acc1.at[pl.ds(zb, ZCH)], cnt1_hbm.at[c].at[pl.ds(zb, ZCH)])
    pltpu.sync_copy(acc2.at[pl.ds(zb, ZCH)], cnt2_hbm.at[c].at[pl.ds(zb, ZCH)])


@functools.partial(
    pl.kernel,
    out_type=(jax.ShapeDtypeStruct((NC, NPAD, D), _f32),
              jax.ShapeDtypeStruct((NC, NPAD), _f32)),
    scratch_types=[
        pltpu.VMEM((KPP, CHUNK), jnp.int32),
        pltpu.VMEM((KPP, CHUNK), jnp.int32),
        pltpu.VMEM((CHUNK,), _f32),
        pltpu.VMEM((CHUNK, D), _f32),
        pltpu.VMEM((CHUNK, D), _f32),
        pltpu.VMEM_SHARED((NPAD, D), _f32),
        pltpu.VMEM_SHARED((NPAD,), _f32),
        pltpu.SemaphoreType.DMA,
        pltpu.SemaphoreType.DMA,
        pltpu.SemaphoreType.DMA,
        pltpu.SemaphoreType.DMA,
        pltpu.SemaphoreType.DMA,
        pltpu.SemaphoreType.DMA,
        pltpu.SemaphoreType.DMA,
    ],
    **_MESH,
)
def _sc_agg(feat_hbm, src_hbm, dst_hbm, z2_hbm, z1_hbm, out_hbm, din_hbm,
            sidx, didx, ones_v, rows0, rows1, acc, dacc,
            isem0, isem1, gsem0, gsem1, ssem0, ssem1, osem):
    c, s, w = _wid()
    rows = (rows0, rows1)
    gsem = (gsem0, gsem1)
    ssem = (ssem0, ssem1)

    pltpu.async_copy(src_hbm.at[w].at[0], sidx, isem0)
    pltpu.async_copy(dst_hbm.at[w].at[0], didx, isem1)
    _fill_ones(ones_v)
    rb = s * ZCH
    pltpu.sync_copy(z2_hbm.at[pl.ds(rb, ZCH)], acc.at[pl.ds(rb, ZCH)])
    pltpu.sync_copy(z1_hbm.at[pl.ds(rb, ZCH)], dacc.at[pl.ds(rb, ZCH)])
    pltpu.make_async_copy(src_hbm.at[w].at[0], sidx, isem0).wait()
    pltpu.make_async_copy(dst_hbm.at[w].at[0], didx, isem1).wait()
    plsc.subcore_barrier()

    def stage(k, b):
        pltpu.make_async_copy(feat_hbm.at[sidx.at[k]], rows[b], gsem[b]).wait()
        pltpu.async_copy(rows[b], acc.at[didx.at[k]], ssem[b], add=True)
        pltpu.async_copy(ones_v, dacc.at[didx.at[k]], osem, add=True)

        @pl.when(k + 2 < KPP)
        def _():
            pltpu.make_async_copy(rows[b], acc.at[didx.at[k]], ssem[b]).wait()
            pltpu.async_copy(feat_hbm.at[sidx.at[k + 2]], rows[b], gsem[b])

    def pair(m, _):
        for b in range(2):
            stage(m * 2 + b, b)
        return ()

    def drain_ones(k, _):
        pltpu.make_async_copy(ones_v, dacc.at[didx.at[0]], osem).wait()
        return ()

    for ph in range(PH):
        for b in range(2):
            pltpu.async_copy(feat_hbm.at[sidx.at[b]], rows[b], gsem[b])
        lax.fori_loop(0, KPP // 2, pair, ())
        for b in range(2):
            pltpu.make_async_copy(rows[b], acc.at[didx.at[0]], ssem[b]).wait()
        lax.fori_loop(0, KPP, drain_ones, ())
        if ph + 1 < PH:
            pltpu.sync_copy(src_hbm.at[w].at[ph + 1], sidx)
            pltpu.sync_copy(dst_hbm.at[w].at[ph + 1], didx)

    plsc.subcore_barrier()
    pltpu.sync_copy(acc.at[pl.ds(rb, ZCH)], out_hbm.at[c].at[pl.ds(rb, ZCH)])
    pltpu.sync_copy(dacc.at[pl.ds(rb, ZCH)], din_hbm.at[c].at[pl.ds(rb, ZCH)])


def _tc1_body(h_ref, n_ref, w_ref, o_ref):
    x = h_ref[...] * n_ref[...]
    o_ref[:N] = jnp.dot(x, w_ref[...], preferred_element_type=_f32)
    o_ref[N:] = jnp.zeros((NPAD - N, D), _f32)


def _tc2_body(p_ref, n_ref, b_ref, pc_ref, w_ref, o_ref):
    agg = p_ref[0, :N] + p_ref[1, :N]
    x = jnp.maximum(agg * n_ref[:, 0:1] + b_ref[...], 0.0)
    x = x * jnp.clip(pc_ref[...], 0.0, 1.0)
    o_ref[:N] = jnp.dot(x * n_ref[:, 1:2], w_ref[...],
                        preferred_element_type=_f32)
    o_ref[N:] = jnp.zeros((NPAD - N, D), _f32)


def _tc3_body(p_ref, n_ref, b_ref, o_ref):
    o_ref[...] = (p_ref[0, :N] + p_ref[1, :N]) * n_ref[...] + b_ref[...]


def _tc1(h, norm, W1):
    return pl.pallas_call(
        _tc1_body, out_shape=jax.ShapeDtypeStruct((NPAD, D), _f32))(h, norm, W1)


def _tc2(part1, norms, b1, p, W2):
    return pl.pallas_call(
        _tc2_body, out_shape=jax.ShapeDtypeStruct((NPAD, D), _f32))(
            part1, norms, b1.reshape(1, D), p.reshape(1, D), W2)


def _tc3(part2, norm, b2):
    return pl.pallas_call(
        _tc3_body, out_shape=jax.ShapeDtypeStruct((N, D), _f32))(
            part2, norm, b2.reshape(1, D))


def _prep_idx(a):
    a = a.astype(jnp.int32)
    pad = jnp.full((EP - E,), N, jnp.int32)
    return jnp.concatenate([a, pad]).reshape(NW, PH, KPP, CHUNK)


def _norm(cnt):
    return lax.rsqrt(jnp.clip(cnt[0, :N] + cnt[1, :N], 1.0, None))


def kernel(h, edge_index1, edge_index2, W1, b1, p, W2, b2):
    src1 = _prep_idx(edge_index1[0])
    dst1 = _prep_idx(edge_index1[1])
    src2 = _prep_idx(edge_index2[0])
    dst2 = _prep_idx(edge_index2[1])

    zeros_n = jnp.zeros((NPAD,), _f32)
    zeros_nd = jnp.zeros((NPAD, D), _f32)

    cnt_o1, cnt_o2 = _sc_degs(src1, src2, zeros_n)
    norm_o1 = _norm(cnt_o1)
    norm_o2 = _norm(cnt_o2)

    feat1 = _tc1(h, norm_o1[:, None], W1)
    part1, cnt_i1 = _sc_agg(feat1, src1, dst1, zeros_nd, zeros_n)
    norms_a = jnp.stack([_norm(cnt_i1), norm_o2], axis=1)
    feat2 = _tc2(part1, norms_a, b1, p, W2)
    part2, cnt_i2 = _sc_agg(feat2, src2, dst2, zeros_nd, zeros_n)
    return _tc3(part2, _norm(cnt_i2)[:, None], b2)
